# baseline (speedup 1.0000x reference)
import functools

import jax
import jax.numpy as jnp
from jax.experimental import pallas as pl
from jax.experimental.pallas import tpu as pltpu

N = 10000
D_IN = 128
D_OUT = 128
BM = 200
SLAB = 2 * BM


def _gcn_body(x_ref, w_ref, b_ref, a0_ref, a1_ref, o_ref, s_ref):
    @pl.when(pl.program_id(0) == 0)
    def _():
        s_ref[...] = jnp.dot(x_ref[...], w_ref[...],
                             preferred_element_type=jnp.float32)

    o_ref[:BM, :] = jnp.dot(a0_ref[...], s_ref[...],
                            preferred_element_type=jnp.float32) + b_ref[...]
    o_ref[BM:, :] = jnp.dot(a1_ref[...], s_ref[...],
                            preferred_element_type=jnp.float32) + b_ref[...]


@functools.partial(jax.jit, static_argnames=())
def kernel(input, adj, W, b):
    b2 = b.reshape(1, D_OUT)
    grid = (N // SLAB,)
    out = pl.pallas_call(
        _gcn_body,
        grid=grid,
        in_specs=[
            pl.BlockSpec((N, D_IN), lambda i: (0, 0)),
            pl.BlockSpec((D_IN, D_OUT), lambda i: (0, 0)),
            pl.BlockSpec((1, D_OUT), lambda i: (0, 0)),
            pl.BlockSpec((BM, N), lambda i: (2 * i, 0)),
            pl.BlockSpec((BM, N), lambda i: (2 * i + 1, 0)),
        ],
        out_specs=pl.BlockSpec((SLAB, D_OUT), lambda i: (i, 0)),
        out_shape=jax.ShapeDtypeStruct((N, D_OUT), jnp.float32),
        scratch_shapes=[pltpu.VMEM((N, D_OUT), jnp.float32)],
        compiler_params=pltpu.CompilerParams(
            dimension_semantics=("arbitrary",),
        ),
    )(input, W, b2, adj, adj)
    return out


# speedup vs baseline: 1.0190x; 1.0190x over previous
import functools

import jax
import jax.numpy as jnp
from jax.experimental import pallas as pl
from jax.experimental.pallas import tpu as pltpu

N = 10000
D_IN = 128
D_OUT = 128
BM = 400


def _gcn_body(x_ref, w_ref, b_ref, adj_ref, o_ref, s_ref):
    @pl.when(pl.program_id(0) == 0)
    def _():
        s_ref[...] = jnp.dot(x_ref[...], w_ref[...],
                             preferred_element_type=jnp.float32)

    acc = jnp.dot(adj_ref[...], s_ref[...],
                  preferred_element_type=jnp.float32)
    o_ref[...] = acc + b_ref[...]


@functools.partial(jax.jit, static_argnames=())
def kernel(input, adj, W, b):
    b2 = b.reshape(1, D_OUT)
    grid = (N // BM,)
    out = pl.pallas_call(
        _gcn_body,
        grid=grid,
        in_specs=[
            pl.BlockSpec((N, D_IN), lambda i: (0, 0)),
            pl.BlockSpec((D_IN, D_OUT), lambda i: (0, 0)),
            pl.BlockSpec((1, D_OUT), lambda i: (0, 0)),
            pl.BlockSpec((BM, N), lambda i: (i, 0)),
        ],
        out_specs=pl.BlockSpec((BM, D_OUT), lambda i: (i, 0)),
        out_shape=jax.ShapeDtypeStruct((N, D_OUT), jnp.float32),
        scratch_shapes=[pltpu.VMEM((N, D_OUT), jnp.float32)],
        compiler_params=pltpu.CompilerParams(
            dimension_semantics=("arbitrary",),
        ),
    )(input, W, b2, adj)
    return out
